# baseline (speedup 1.0000x reference)
import functools

import jax
import jax.numpy as jnp
from jax import lax
from jax.experimental import pallas as pl
from jax.experimental.pallas import tpu as pltpu
from jax.experimental.pallas import tpu_sc as plsc

EMB = 64
CHUNK = 128


def _make_emb_kernel(n_total, n_chunks_per_worker, nc, ns):
    nw = nc * ns
    per_worker = n_total // nw
    mesh = plsc.VectorSubcoreMesh(core_axis_name="c", subcore_axis_name="s")

    @functools.partial(
        pl.kernel,
        mesh=mesh,
        out_type=jax.ShapeDtypeStruct((n_total, EMB), jnp.float32),
        compiler_params=pltpu.CompilerParams(use_tc_tiling_on_sc=False),
        scratch_types=[
            pltpu.VMEM((n_chunks_per_worker, CHUNK), jnp.int32),
            pltpu.VMEM((CHUNK, EMB), jnp.float32),
            pltpu.VMEM((CHUNK, EMB), jnp.float32),
            pltpu.SemaphoreType.DMA,
            pltpu.SemaphoreType.DMA,
            pltpu.SemaphoreType.DMA,
            pltpu.SemaphoreType.DMA,
        ],
    )
    def emb_kernel(x_hbm, table_hbm, out_hbm, idx_v, rows0, rows1, g0, g1, s0, s1):
        wid = lax.axis_index("s") * nc + lax.axis_index("c")
        base = wid * per_worker

        pltpu.sync_copy(x_hbm.at[wid], idx_v)

        rows = (rows0, rows1)
        gsem = (g0, g1)
        ssem = (s0, s1)

        def gather_start(chunk, buf):
            pltpu.make_async_copy(
                table_hbm.at[idx_v.at[chunk]], rows[buf], gsem[buf]
            ).start()

        def gather_wait(buf):
            pltpu.make_async_copy(
                table_hbm.at[idx_v.at[0]], rows[buf], gsem[buf]
            ).wait()

        def store_start(chunk, buf):
            row = base + chunk * CHUNK
            pltpu.make_async_copy(
                rows[buf], out_hbm.at[pl.ds(row, CHUNK)], ssem[buf]
            ).start()

        def store_wait(buf):
            pltpu.make_async_copy(
                rows[buf], out_hbm.at[pl.ds(base, CHUNK)], ssem[buf]
            ).wait()

        gather_start(0, 0)
        gather_start(1, 1)

        def body(c, _):
            def on_buf(b):
                gather_wait(b)
                store_start(c, b)
                store_wait(b)

                @pl.when(c + 2 < n_chunks_per_worker)
                def _():
                    gather_start(c + 2, b)

            lax.cond(lax.rem(c, 2) == 0, lambda: on_buf(0), lambda: on_buf(1))
            return 0

        lax.fori_loop(0, n_chunks_per_worker, body, 0)

    return emb_kernel


def kernel(x, table):
    b, l = x.shape
    n_total = b * l
    info = plsc.get_sparse_core_info()
    nc, ns = info.num_cores, info.num_subcores
    nw = nc * ns
    per_worker = n_total // nw
    n_chunks = per_worker // CHUNK

    xf = x.reshape(nw, n_chunks, CHUNK).astype(jnp.int32)
    emb = _make_emb_kernel(n_total, n_chunks, nc, ns)
    out = emb(xf, table)
    return out.reshape(b, l, EMB)


# speedup vs baseline: 1.0207x; 1.0207x over previous
import functools

import jax
import jax.numpy as jnp
from jax import lax
from jax.experimental import pallas as pl
from jax.experimental.pallas import tpu as pltpu
from jax.experimental.pallas import tpu_sc as plsc

EMB = 64
CHUNK = 128
GROUP = 2
NBUF = 4
ROWS = CHUNK * GROUP


def _make_emb_kernel(n_total, n_chunks_per_worker, nc, ns):
    nw = nc * ns
    per_worker = n_total // nw
    n_groups = n_chunks_per_worker // GROUP
    mesh = plsc.VectorSubcoreMesh(core_axis_name="c", subcore_axis_name="s")

    @functools.partial(
        pl.kernel,
        mesh=mesh,
        out_type=jax.ShapeDtypeStruct((n_total, EMB), jnp.float32),
        compiler_params=pltpu.CompilerParams(use_tc_tiling_on_sc=False),
        scratch_types=(
            [pltpu.VMEM((n_chunks_per_worker, CHUNK), jnp.int32)]
            + [pltpu.VMEM((ROWS, EMB), jnp.float32)] * NBUF
            + [pltpu.SemaphoreType.DMA] * (2 * NBUF)
        ),
    )
    def emb_kernel(x_hbm, table_hbm, out_hbm, idx_v, *bufs_and_sems):
        rows = bufs_and_sems[:NBUF]
        gsem = bufs_and_sems[NBUF : 2 * NBUF]
        ssem = bufs_and_sems[2 * NBUF :]

        wid = lax.axis_index("s") * nc + lax.axis_index("c")
        base = wid * per_worker

        pltpu.sync_copy(x_hbm.at[wid], idx_v)

        def gather_start(group, buf):
            for j in range(GROUP):
                pltpu.make_async_copy(
                    table_hbm.at[idx_v.at[group * GROUP + j]],
                    rows[buf].at[pl.ds(j * CHUNK, CHUNK)],
                    gsem[buf],
                ).start()

        def gather_wait(buf):
            pltpu.make_async_copy(table_hbm.at[idx_v.at[0]], rows[buf], gsem[buf]).wait()

        def store_start(group, buf):
            pltpu.make_async_copy(
                rows[buf], out_hbm.at[pl.ds(base + group * ROWS, ROWS)], ssem[buf]
            ).start()

        def store_wait(buf):
            pltpu.make_async_copy(
                rows[buf], out_hbm.at[pl.ds(base, ROWS)], ssem[buf]
            ).wait()

        for b in range(NBUF):
            gather_start(b, b)

        n_main = n_groups // NBUF - 1

        def body(i, _):
            for b in range(NBUF):
                g = i * NBUF + b
                gather_wait(b)
                store_start(g, b)
                store_wait(b)
                gather_start(g + NBUF, b)
            return 0

        lax.fori_loop(0, n_main, body, 0)

        for b in range(NBUF):
            g = n_main * NBUF + b
            gather_wait(b)
            store_start(g, b)
            store_wait(b)

    return emb_kernel


def kernel(x, table):
    b, l = x.shape
    n_total = b * l
    info = plsc.get_sparse_core_info()
    nc, ns = info.num_cores, info.num_subcores
    nw = nc * ns
    per_worker = n_total // nw
    n_chunks = per_worker // CHUNK

    xf = x.reshape(nw, n_chunks, CHUNK).astype(jnp.int32)
    emb = _make_emb_kernel(n_total, n_chunks, nc, ns)
    out = emb(xf, table)
    return out.reshape(b, l, EMB)


# speedup vs baseline: 1.3539x; 1.3265x over previous
import functools

import jax
import jax.numpy as jnp
from jax import lax
from jax.experimental import pallas as pl
from jax.experimental.pallas import tpu as pltpu
from jax.experimental.pallas import tpu_sc as plsc

EMB = 64
PAD = 128
NBUF = 4


def _make_emb_kernel(b_total, l_seq, nc, ns):
    nw = nc * ns
    rows_per_w = b_total // nw
    c0 = min(128, l_seq)
    c1 = l_seq - c0
    mesh = plsc.VectorSubcoreMesh(core_axis_name="c", subcore_axis_name="s")

    @functools.partial(
        pl.kernel,
        mesh=mesh,
        out_type=jax.ShapeDtypeStruct((b_total, l_seq, PAD), jnp.float32),
        compiler_params=pltpu.CompilerParams(use_tc_tiling_on_sc=False),
        scratch_types=(
            [pltpu.VMEM((rows_per_w, l_seq), jnp.int32)]
            + [pltpu.VMEM((l_seq, EMB), jnp.float32)] * NBUF
            + [pltpu.SemaphoreType.DMA] * (2 * NBUF)
        ),
    )
    def emb_kernel(x_hbm, table_hbm, out_hbm, idx_v, *bufs_and_sems):
        rows = bufs_and_sems[:NBUF]
        gsem = bufs_and_sems[NBUF : 2 * NBUF]
        ssem = bufs_and_sems[2 * NBUF :]

        wid = lax.axis_index("s") * nc + lax.axis_index("c")
        base = wid * rows_per_w

        pltpu.sync_copy(x_hbm.at[pl.ds(base, rows_per_w)], idx_v)

        def gather_start(bi, buf):
            pltpu.make_async_copy(
                table_hbm.at[idx_v.at[bi, pl.ds(0, c0)]],
                rows[buf].at[pl.ds(0, c0)],
                gsem[buf],
            ).start()
            if c1:
                pltpu.make_async_copy(
                    table_hbm.at[idx_v.at[bi, pl.ds(c0, c1)]],
                    rows[buf].at[pl.ds(c0, c1)],
                    gsem[buf],
                ).start()

        def gather_wait(buf):
            pltpu.make_async_copy(
                table_hbm.at[idx_v.at[0, pl.ds(0, l_seq)]], rows[buf], gsem[buf]
            ).wait()

        def store_start(bi, buf):
            pltpu.make_async_copy(
                rows[buf], out_hbm.at[base + bi, :, pl.ds(0, EMB)], ssem[buf]
            ).start()

        def store_wait(buf):
            pltpu.make_async_copy(
                rows[buf], out_hbm.at[base, :, pl.ds(0, EMB)], ssem[buf]
            ).wait()

        for b in range(NBUF):
            gather_start(b, b)

        n_main = rows_per_w // NBUF - 1

        def body(i, _):
            for b in range(NBUF):
                bi = i * NBUF + b
                gather_wait(b)
                store_start(bi, b)
                store_wait(b)
                gather_start(bi + NBUF, b)
            return 0

        lax.fori_loop(0, n_main, body, 0)

        for b in range(NBUF):
            bi = n_main * NBUF + b
            gather_wait(b)
            store_start(bi, b)
            store_wait(b)

    return emb_kernel


def kernel(x, table):
    b, l = x.shape
    info = plsc.get_sparse_core_info()
    nc, ns = info.num_cores, info.num_subcores
    emb = _make_emb_kernel(b, l, nc, ns)
    out = emb(x.astype(jnp.int32), table)
    return out[:, :, :EMB]
